# Initial kernel scaffold; baseline (speedup 1.0000x reference)
import jax
import jax.numpy as jnp
from jax.experimental import pallas as pl


def kernel(users, items, user_emb, item_emb, ui_row, ui_col, ui_val, ii_row, ii_col, ii_val):
    raise NotImplementedError("write your pallas kernel here")


# speedup vs baseline: 2.0047x; 2.0047x over previous
import functools
import jax
import jax.numpy as jnp
from jax import lax
from jax.experimental import pallas as pl
from jax.experimental.pallas import tpu as pltpu
from jax.experimental.pallas import tpu_sc as plsc

N_USERS = 5000
N_ITEMS = 5000
D = 256
HALF = 128
NTOT = N_USERS + N_ITEMS
UI_PAD = 10240
II_PAD = 5120
BATCH = 4096

NC = 2
NS = 16
CH = 128

ACC_ROWS = 10112
ACC_CHUNKS = ACC_ROWS // 128

UI_E_T = 10240
II_E_T = 5120

W_UI = 0.25
W_SEM = 1.0 / 3.0
SEM_W = 0.5


def _scale_rows(rows_ref, val_ref, n):
    def body(e, carry):
        v = plsc.load_gather(val_ref, [jnp.full((16,), e, jnp.int32)])
        for j in range(HALF // 16):
            sl = pl.ds(j * 16, 16)
            rows_ref[e, sl] = rows_ref[e, sl] * v
        return carry
    lax.fori_loop(0, n, body, 0)


def _sc_body(emb0h, ui_gidx, ui_ridx, ui_val, ii_gidx1, ii_gidx2, ii_ridx,
             ii_val, uq, iq_ui, iq_ii,
             ui1, ui2, ui3, ii1, ii2, gamma,
             accum, gidx_v, ridx_v, val_v, rows_v, uacc, iacc,
             qidx_v, gout, sem):
    c = lax.axis_index("c")
    s = lax.axis_index("s")

    def zero_rows_v():
        def zb(e, carry):
            for j in range(HALF // 16):
                rows_v[e, pl.ds(j * 16, 16)] = jnp.zeros((16,), jnp.float32)
            return carry
        lax.fori_loop(0, CH, zb, 0)

    def spmm_phase(src_tbl, gidx_hbm, ridx_hbm, val_hbm, e_t, n_chunks,
                   dst_tbl, acc_chunks, dst_stride):
        zero_rows_v()
        for k in range((acc_chunks + NS - 1) // NS):
            cid = s + NS * k
            @pl.when(cid < acc_chunks)
            def _():
                pltpu.sync_copy(rows_v, accum.at[pl.ds(cid * CH, CH)])
        plsc.subcore_barrier()

        def chunk(k, carry):
            base = s * e_t + k * CH
            pltpu.sync_copy(gidx_hbm.at[c, pl.ds(base, CH)], gidx_v)
            pltpu.sync_copy(ridx_hbm.at[pl.ds(base, CH)], ridx_v)
            pltpu.sync_copy(val_hbm.at[pl.ds(base, CH)], val_v)
            pltpu.async_copy(src_tbl.at[gidx_v], rows_v, sem).wait()
            _scale_rows(rows_v, val_v, CH)
            pltpu.sync_copy(rows_v, accum.at[ridx_v], add=True)
            return carry
        lax.fori_loop(0, n_chunks, chunk, 0)
        plsc.subcore_barrier()
        for k in range((acc_chunks + NS - 1) // NS):
            cid = s + NS * k
            @pl.when(cid < acc_chunks)
            def _():
                pltpu.sync_copy(accum.at[pl.ds(cid * CH, CH)], rows_v)
                pltpu.sync_copy(rows_v,
                                dst_tbl.at[pl.ds(c * dst_stride + cid * CH, CH)])
        plsc.subcore_barrier()

    spmm_phase(emb0h, ui_gidx, ui_ridx, ui_val, UI_E_T, UI_E_T // CH,
               ui1, ACC_CHUNKS, UI_PAD)
    spmm_phase(ui1, ui_gidx, ui_ridx, ui_val, UI_E_T, UI_E_T // CH,
               ui2, ACC_CHUNKS, UI_PAD)
    spmm_phase(ui2, ui_gidx, ui_ridx, ui_val, UI_E_T, UI_E_T // CH,
               ui3, ACC_CHUNKS, UI_PAD)
    spmm_phase(emb0h, ii_gidx1, ii_ridx, ii_val, II_E_T, II_E_T // CH,
               ii1, II_PAD // CH, II_PAD)
    spmm_phase(ii1, ii_gidx2, ii_ridx, ii_val, II_E_T, II_E_T // CH,
               ii2, II_PAD // CH, II_PAD)

    u_tables = [(emb0h, uq, W_UI), (ui1, uq, W_UI), (ui2, uq, W_UI),
                (ui3, uq, W_UI)]
    i_tables = [(emb0h, iq_ui, (1.0 - SEM_W) * W_UI + SEM_W * W_SEM),
                (ui1, iq_ui, (1.0 - SEM_W) * W_UI),
                (ui2, iq_ui, (1.0 - SEM_W) * W_UI),
                (ui3, iq_ui, (1.0 - SEM_W) * W_UI),
                (ii1, iq_ii, SEM_W * W_SEM),
                (ii2, iq_ii, SEM_W * W_SEM)]

    def gather_acc(acc_ref, tables, base):
        first = True
        for tbl, idx_hbm, w in tables:
            pltpu.sync_copy(idx_hbm.at[c, pl.ds(base, CH)], qidx_v)
            pltpu.async_copy(tbl.at[qidx_v], rows_v, sem).wait()
            is_first = first

            def acc_body(e, carry):
                for j in range(HALF // 16):
                    sl = pl.ds(j * 16, 16)
                    v = rows_v[e, sl] * w
                    if is_first:
                        acc_ref[e, sl] = v
                    else:
                        plsc.addupdate(acc_ref.at[e, sl], v)
                return carry
            lax.fori_loop(0, CH, acc_body, 0)
            first = False

    for chunk_i in range(BATCH // NS // CH):
        base = s * (BATCH // NS) + chunk_i * CH
        gather_acc(uacc, u_tables, base)
        gather_acc(iacc, i_tables, base)

        lane = lax.iota(jnp.int32, 16)

        def dot_body(e, carry):
            acc = jnp.zeros((16,), jnp.float32)
            for j in range(HALF // 16):
                sl = pl.ds(j * 16, 16)
                acc = acc + uacc[e, sl] * iacc[e, sl]
            s_val = jnp.sum(acc)
            plsc.store_scatter(gout, [jnp.full((16,), e, jnp.int32)],
                               jnp.broadcast_to(s_val, (16,)),
                               mask=lane == 0)
            return carry
        lax.fori_loop(0, CH, dot_body, 0)
        pltpu.sync_copy(gout, gamma.at[c, pl.ds(base, CH)])


def _pad_per_tile(arr, e_t, fill):
    per = arr.shape[0] // NS
    a = arr.reshape(NS, per)
    pad = jnp.full((NS, e_t - per), fill, a.dtype)
    return jnp.concatenate([a, pad], axis=1).reshape(-1)


@jax.jit
def kernel(users, items, user_emb, item_emb, ui_row, ui_col, ui_val,
           ii_row, ii_col, ii_val):
    all_emb = jnp.concatenate([user_emb, item_emb], axis=0)
    zpad = jnp.zeros((UI_PAD - NTOT, HALF), jnp.float32)
    emb0h = jnp.concatenate(
        [all_emb[:, :HALF], zpad, all_emb[:, HALF:], zpad], axis=0)

    core_off = jnp.array([[0], [UI_PAD]], jnp.int32)
    core_off_ii = jnp.array([[0], [II_PAD]], jnp.int32)

    ui_col_p = _pad_per_tile(ui_col, UI_E_T, 0)
    ui_row_p = _pad_per_tile(ui_row, UI_E_T, 0)
    ui_val_p = _pad_per_tile(ui_val, UI_E_T, 0.0)
    ii_col_p = _pad_per_tile(ii_col, II_E_T, 0)
    ii_row_p = _pad_per_tile(ii_row, II_E_T, 0)
    ii_val_p = _pad_per_tile(ii_val, II_E_T, 0.0)

    ui_gidx = ui_col_p[None, :] + core_off
    ii_gidx1 = ii_col_p[None, :] + N_USERS + core_off
    ii_gidx2 = ii_col_p[None, :] + core_off_ii
    uq = users[None, :] + core_off
    iq_ui = items[None, :] + N_USERS + core_off
    iq_ii = items[None, :] + core_off_ii

    mesh = plsc.VectorSubcoreMesh(core_axis_name="c", subcore_axis_name="s")
    f32 = jnp.float32
    out_type = [
        jax.ShapeDtypeStruct((NC * UI_PAD, HALF), f32),
        jax.ShapeDtypeStruct((NC * UI_PAD, HALF), f32),
        jax.ShapeDtypeStruct((NC * UI_PAD, HALF), f32),
        jax.ShapeDtypeStruct((NC * II_PAD, HALF), f32),
        jax.ShapeDtypeStruct((NC * II_PAD, HALF), f32),
        jax.ShapeDtypeStruct((NC, BATCH), f32),
    ]
    scratch = [
        pltpu.VMEM_SHARED((ACC_ROWS, HALF), f32),
        pltpu.VMEM((CH,), jnp.int32),
        pltpu.VMEM((CH,), jnp.int32),
        pltpu.VMEM((CH,), f32),
        pltpu.VMEM((CH, HALF), f32),
        pltpu.VMEM((CH, HALF), f32),
        pltpu.VMEM((CH, HALF), f32),
        pltpu.VMEM((CH,), jnp.int32),
        pltpu.VMEM((CH,), f32),
        pltpu.SemaphoreType.DMA,
    ]
    run = pl.kernel(_sc_body, out_type=out_type, mesh=mesh,
                    scratch_types=scratch,
                    compiler_params=pltpu.CompilerParams(
                        needs_layout_passes=False))
    outs = run(emb0h, ui_gidx, ui_row_p, ui_val_p, ii_gidx1, ii_gidx2,
               ii_row_p, ii_val_p, uq, iq_ui, iq_ii)
    gamma_halves = outs[5]
    return gamma_halves[0] + gamma_halves[1]
